# Initial kernel scaffold; baseline (speedup 1.0000x reference)
import jax
import jax.numpy as jnp
from jax.experimental import pallas as pl


def kernel(r, d, ws, bs, wp, bp, w1, b1, w2, we, g_scale, g_bias, w3, bn3_s, bn3_b, w5, bn5_s, bn5_b):
    raise NotImplementedError("write your pallas kernel here")


# speedup vs baseline: 3.7956x; 3.7956x over previous
import functools

import jax
import jax.numpy as jnp
import numpy as np
from jax.experimental import pallas as pl
from jax.experimental.pallas import tpu as pltpu


def _lowpass_matrix(n, cutoff_ratio=0.25):
    r = max(int(n * cutoff_ratio), 1)
    shifted = (np.arange(n) + n // 2) % n
    mask = (np.abs(shifted - n // 2) <= r).astype(np.float64)
    eye = np.eye(n)
    B = np.fft.ifft(mask[:, None] * np.fft.fft(eye, axis=0), axis=0).real
    return jnp.asarray(B, jnp.float32)


def _sep_masks(H, W):
    yy, xx = np.meshgrid(np.arange(H), np.arange(W), indexing='ij')
    ym = np.zeros((8, H * W), np.float32)
    xm = np.zeros((8, H * W), np.float32)
    for i, dlt in enumerate(range(-2, 3)):
        ym[i] = ((yy + dlt >= 0) & (yy + dlt < H)).reshape(-1)
        xm[i] = ((xx + dlt >= 0) & (xx + dlt < W)).reshape(-1)
    return (jnp.asarray(ym, jnp.bfloat16), jnp.asarray(xm, jnp.bfloat16))


def _gbf_body(C, H, W,
              r_ref, d_ref, ym_ref, xm_ref,
              wsp_ref, bsp_ref, w1t_ref, b1_ref, w2_ref, we_ref,
              gs_ref, gb_ref, wconv_ref, s5_ref, b5_ref, s3_ref, b3_ref,
              r_out_ref, s_out_ref, d_out_ref,
              dy_ref, tap_ref):
    HW = H * W
    wsp, bsp = wsp_ref[...], bsp_ref[...]
    w1t, b1 = w1t_ref[...], b1_ref[...]
    w2, we = w2_ref[...], we_ref[...]
    g_scale, g_bias = gs_ref[...], gb_ref[...]

    def glore(x):
        y = jnp.dot(wsp, x, preferred_element_type=jnp.float32) + bsp
        xs = y[:2 * C, :]
        xp = y[2 * C:, :]
        xn = jax.lax.dot_general(
            xs, xp, (((1,), (1,)), ((), ())),
            preferred_element_type=jnp.float32)
        h = jnp.dot(xn, w1t, preferred_element_type=jnp.float32) + b1
        h = jnp.maximum(h + xn, 0.0)
        xrel = jnp.dot(w2, h, preferred_element_type=jnp.float32)
        wx = jnp.dot(we, xrel, preferred_element_type=jnp.float32)
        ext = jnp.dot(wx, xp, preferred_element_type=jnp.float32)
        return x + ext * g_scale + g_bias

    r = glore(r_ref[0].astype(jnp.float32))
    d = glore(d_ref[0].astype(jnp.float32))
    s16 = (r + d).astype(jnp.bfloat16)

    for i, dy in enumerate(range(-2, 3)):
        shift = (-(dy * W)) % HW
        t = pltpu.roll(s16, shift, 1) if shift else s16
        if dy:
            t = t * ym_ref[pl.ds(i, 1), :]
        dy_ref[pl.ds(i * C, C), :] = t

    stack = dy_ref[...]
    for j, dx in enumerate(range(-2, 3)):
        shift = (-dx) % HW
        t = pltpu.roll(stack, shift, 1) if shift else stack
        if dx:
            t = t * xm_ref[pl.ds(j, 1), :]
        tap_ref[pl.ds(j * 5 * C, 5 * C), :] = t

    acc = jnp.dot(wconv_ref[...], tap_ref[...],
                  preferred_element_type=jnp.float32)
    t5 = jnp.maximum(acc[:C] * s5_ref[...] + b5_ref[...], 0.0)
    t3 = jnp.maximum(acc[C:] * s3_ref[...] + b3_ref[...], 0.0)
    s = t5 - t3

    r_out_ref[0] = s + r
    s_out_ref[0] = s
    d_out_ref[0] = d


def kernel(r, d, ws, bs, wp, bp, w1, b1, w2, we, g_scale, g_bias,
           w3, bn3_s, bn3_b, w5, bn5_s, bn5_b):
    N, C, H, W = r.shape
    HW = H * W
    inv_hw = 1.0 / float(HW)

    ym, xm = _sep_masks(H, W)

    wsp = jnp.concatenate([ws * inv_hw, wp], axis=0)
    bsp = jnp.concatenate([bs * inv_hw, bp], axis=0).reshape(3 * C, 1)

    wb5 = jnp.transpose(w5, (0, 3, 2, 1))
    wb3 = jnp.zeros((C, 5, 5, C), jnp.float32)
    wb3 = wb3.at[:, 1:4, 1:4, :].set(jnp.transpose(w3, (0, 3, 2, 1)))
    wconv = jnp.concatenate([wb5.reshape(C, 25 * C),
                             wb3.reshape(C, 25 * C)], axis=0)
    wconv = wconv.astype(jnp.bfloat16)

    const_args = [
        ym, xm,
        wsp, bsp,
        w1.T, b1.reshape(1, C),
        w2, we,
        g_scale.reshape(C, 1), g_bias.reshape(C, 1),
        wconv,
        bn5_s.reshape(C, 1), bn5_b.reshape(C, 1),
        bn3_s.reshape(C, 1), bn3_b.reshape(C, 1),
    ]

    img_spec = pl.BlockSpec((1, C, HW), lambda b: (b, 0, 0))

    def const_spec(a):
        idx = (0,) * a.ndim
        return pl.BlockSpec(a.shape, lambda b, _idx=idx: _idx)

    r_out, s_out, d_gl = pl.pallas_call(
        functools.partial(_gbf_body, C, H, W),
        out_shape=(jax.ShapeDtypeStruct((N, C, HW), jnp.float32),) * 3,
        grid_spec=pltpu.PrefetchScalarGridSpec(
            num_scalar_prefetch=0,
            grid=(N,),
            in_specs=[img_spec, img_spec] + [const_spec(a) for a in const_args],
            out_specs=(img_spec,) * 3,
            scratch_shapes=[pltpu.VMEM((5 * C, HW), jnp.bfloat16),
                            pltpu.VMEM((25 * C, HW), jnp.bfloat16)]),
        compiler_params=pltpu.CompilerParams(dimension_semantics=("parallel",)),
    )(r.reshape(N, C, HW), d.reshape(N, C, HW), *const_args)

    By = _lowpass_matrix(H)
    Bx = _lowpass_matrix(W)
    d4 = d_gl.reshape(N, C, H, W)
    d_f = jnp.einsum('ncpw,wq->ncpq',
                     jnp.einsum('nchw,hp->ncpw', d4, By), Bx)
    r_final = r_out.reshape(N, C, H, W)
    d_final = s_out.reshape(N, C, H, W) + d_f
    return r_final, d_final
